# baseline (speedup 1.0000x reference)
import functools

import jax
import jax.numpy as jnp
from jax.experimental import pallas as pl
from jax.experimental.pallas import tpu as pltpu
from jax.experimental.pallas import tpu_sc as plsc

_B = 16384
_W = 128
_F32 = jnp.float32


def _sc_gather_pair(ua, ib, tab_a, tab_b):
    mesh = plsc.VectorSubcoreMesh(core_axis_name="c", subcore_axis_name="s")
    out_type = [jax.ShapeDtypeStruct((_B, 128), _F32) for _ in range(2)]

    @functools.partial(pl.kernel, mesh=mesh, out_type=out_type)
    def k(a_hbm, b_hbm, ua_hbm, ib_hbm, o_a, o_b):
        def body(ua_v, ib_v, ba, bb):
            pltpu.sync_copy(a_hbm.at[ua_v.at[0]], ba)
            pltpu.sync_copy(b_hbm.at[ib_v.at[0]], bb)

        pltpu.emit_pipeline(
            body,
            grid=(_B // _W,),
            in_specs=[pl.BlockSpec((1, _W), lambda i: (0, i))] * 2,
            out_specs=[pl.BlockSpec((_W, 128), lambda i: (i, 0))] * 2,
            core_axis_name=("c", "s"),
            dimension_semantics=(pltpu.PARALLEL,),
        )(ua_hbm, ib_hbm, o_a, o_b)

    return k(tab_a, tab_b, ua, ib)


_BLK = 2048


def _sel4(g, q):
    return jnp.where(
        q < 2,
        jnp.where(q == 0, g[:, 0:32], g[:, 32:64]),
        jnp.where(q == 2, g[:, 64:96], g[:, 96:128]),
    )


def _sel16(g, q):
    out = g[:, 0:8]
    for s in range(1, 16):
        out = jnp.where(q == s, g[:, 8 * s:8 * s + 8], out)
    return out


def _tc_mlp_body(gnnu_ref, gnni_ref, gmfu_ref, gmfi_ref,
                 qu_ref, qi_ref, su_ref, si_ref,
                 W1_ref, b1_ref, W2_ref, b2_ref, W3_ref, b3_ref,
                 W4_ref, b4_ref, out_ref):
    dot = lambda a, b: jax.lax.dot_general(
        a, b, (((1,), (0,)), ((), ())), preferred_element_type=_F32)
    nnu = _sel4(gnnu_ref[...], qu_ref[...])
    nni = _sel4(gnni_ref[...], qi_ref[...])
    mfu = _sel16(gmfu_ref[...], su_ref[...])
    mfi = _sel16(gmfi_ref[...], si_ref[...])
    w1 = W1_ref[...]
    x = dot(nnu, w1[:32]) + dot(nni, w1[32:]) + b1_ref[...]
    x = jnp.maximum(x, 0.0)
    x = dot(x, W2_ref[...]) + b2_ref[...]
    x = jnp.maximum(x, 0.0)
    x = dot(x, W3_ref[...]) + b3_ref[...]
    x = jnp.maximum(x, 0.0)
    w4 = W4_ref[...]
    out_ref[...] = (dot(mfu * mfi, w4[:8]) + dot(x, w4[8:]) + b4_ref[...])


def _tc_mlp(gnnu, gnni, gmfu, gmfi, qu, qi, su, si,
            W1, b1, W2, b2, W3, b3, W4, b4):
    grid = (_B // _BLK,)
    bspec = lambda c: pl.BlockSpec((_BLK, c), lambda i: (i, 0))
    wspec = lambda r, c: pl.BlockSpec((r, c), lambda i: (0, 0))
    return pl.pallas_call(
        _tc_mlp_body,
        grid=grid,
        in_specs=[
            bspec(128), bspec(128), bspec(128), bspec(128),
            bspec(1), bspec(1), bspec(1), bspec(1),
            wspec(64, 32), wspec(1, 32), wspec(32, 16), wspec(1, 16),
            wspec(16, 8), wspec(1, 8), wspec(16, 5), wspec(1, 5),
        ],
        out_specs=pl.BlockSpec((_BLK, 5), lambda i: (i, 0)),
        out_shape=jax.ShapeDtypeStruct((_B, 5), _F32),
    )(gnnu, gnni, gmfu, gmfi, qu, qi, su, si,
      W1, b1.reshape(1, 32), W2, b2.reshape(1, 16),
      W3, b3.reshape(1, 8), W4, b4.reshape(1, 5))


def kernel(user, item, mf_usr_emb, mf_item_emb, nn_usr_emb, nn_item_emb,
           W1, b1, W2, b2, W3, b3, W4, b4):
    user = user.astype(jnp.int32)
    item = item.astype(jnp.int32)
    r_nnu = nn_usr_emb.reshape(250000, 128)
    r_nni = nn_item_emb.reshape(250000, 128)
    r_mfu = mf_usr_emb.reshape(62500, 128)
    r_mfi = mf_item_emb.reshape(62500, 128)
    uh = (user >> 2).reshape(1, _B)
    ih = (item >> 2).reshape(1, _B)
    um = (user >> 4).reshape(1, _B)
    im = (item >> 4).reshape(1, _B)
    gnnu, gnni = _sc_gather_pair(uh, ih, r_nnu, r_nni)
    gmfu, gmfi = _sc_gather_pair(um, im, r_mfu, r_mfi)
    qu = (user & 3).reshape(_B, 1)
    qi = (item & 3).reshape(_B, 1)
    su = (user & 15).reshape(_B, 1)
    si = (item & 15).reshape(_B, 1)
    return _tc_mlp(gnnu, gnni, gmfu, gmfi, qu, qi, su, si,
                   W1, b1, W2, b2, W3, b3, W4, b4)
